# baseline (speedup 1.0000x reference)
import functools

import jax
import jax.numpy as jnp
from jax import lax
from jax.experimental import pallas as pl
from jax.experimental.pallas import tpu as pltpu
from jax.experimental.pallas import tpu_sc as plsc

_MASK_RATIO = 0.5
_NEIGH_WEIGHT = 0.5

_NC = 2
_NS = 16
_NW = _NC * _NS
_CH = 128
_ACC = 10240
_RPT = _ACC // _NS


def _sc_body(nchunk, d, idx_ref, emb_ref, acc_out, deg_out,
             idx_g, buf, zrow, ones_l, zdeg, acc_s, deg_r_s, deg_c_s,
             gsems, isems):
    cid = lax.axis_index("c")
    sid = lax.axis_index("s")
    tid = cid * _NS + sid

    z16 = jnp.zeros((16,), jnp.float32)
    o16 = jnp.full((16,), 1.0, jnp.float32)
    for i in range(16):
        for j in range(d // 16):
            zrow[i, pl.ds(j * 16, 16)] = z16
    for j in range(_CH // 16):
        ones_l[pl.ds(j * 16, 16)] = o16

    def _zero_zdeg(k, carry):
        zdeg[pl.ds(k * 16, 16)] = z16
        return carry

    lax.fori_loop(0, _RPT // 16, _zero_zdeg, 0)

    base = sid * _RPT
    pltpu.sync_copy(zdeg, deg_r_s.at[pl.ds(base, _RPT)])
    pltpu.sync_copy(zdeg, deg_c_s.at[pl.ds(base, _RPT)])
    for b in range(_RPT // 16):
        pltpu.sync_copy(zrow, acc_s.at[pl.ds(base + b * 16, 16)])
    plsc.subcore_barrier()

    pltpu.sync_copy(idx_ref.at[tid, 0], idx_g.at[0])
    pltpu.async_copy(emb_ref.at[idx_g.at[0, 1]], buf.at[0], gsems[0])

    def _step(i, p):
        c = 2 * i + p
        q = 1 - p

        @pl.when(c + 1 < nchunk)
        def _():
            pltpu.async_copy(idx_ref.at[tid, c + 1], idx_g.at[q], isems[q])

        pltpu.sync_copy(ones_l, deg_r_s.at[idx_g.at[p, 0]], add=True)
        pltpu.sync_copy(ones_l, deg_c_s.at[idx_g.at[p, 1]], add=True)
        pltpu.make_async_copy(emb_ref.at[idx_g.at[0, 1]], buf.at[p],
                              gsems[p]).wait()

        @pl.when(c + 1 < nchunk)
        def _():
            pltpu.make_async_copy(idx_ref.at[tid, 0], idx_g.at[q],
                                  isems[q]).wait()
            pltpu.async_copy(emb_ref.at[idx_g.at[q, 1]], buf.at[q], gsems[q])

        pltpu.sync_copy(buf.at[p], acc_s.at[idx_g.at[p, 0]], add=True)

    def body(i, carry):
        for p in range(2):
            _step(i, p)
        return carry

    lax.fori_loop(0, nchunk // 2, body, 0)
    plsc.subcore_barrier()

    pltpu.sync_copy(acc_s.at[pl.ds(base, _RPT)],
                    acc_out.at[pl.ds(cid * _ACC + base, _RPT)])
    pltpu.sync_copy(deg_r_s.at[pl.ds(base, _RPT)],
                    deg_out.at[cid, 0, pl.ds(base, _RPT)])
    pltpu.sync_copy(deg_c_s.at[pl.ds(base, _RPT)],
                    deg_out.at[cid, 1, pl.ds(base, _RPT)])


def _tc_body(num_mask, num_edges, emb_ref, acc_ref, deg_ref, mvec_ref, wt_ref,
             out_ref):
    emb = emb_ref[...]
    ns = acc_ref[0] + acc_ref[1]
    dr_raw = deg_ref[0, 0] + deg_ref[1, 0]
    dc = deg_ref[0, 1] + deg_ref[1, 1]
    mvec = mvec_ref[...]

    r = jnp.dot(emb, wt_ref[...], preferred_element_type=jnp.float32)
    nm = ns / jnp.maximum(dr_raw, 1.0)[:, None]
    dvec = r - nm
    recon_sum = jnp.sum(mvec * jnp.sum(dvec * dvec, axis=1))
    nrm = jnp.sum(emb * emb, axis=1)
    sq_sum = jnp.sum((dr_raw + dc) * nrm)
    dot_sum = jnp.sum(emb * ns)

    d = emb.shape[1]
    recon_loss = recon_sum / (num_mask * d)
    neigh_loss = (sq_sum - 2.0 * dot_sum) / num_edges
    total = recon_loss + _NEIGH_WEIGHT * neigh_loss
    out_ref[...] = total[None, None]


def kernel(x, edge_index, embed):
    n, d = embed.shape
    e = edge_index.shape[1]
    num_mask = max(1, int(_MASK_RATIO * n))

    rkey = jax.random.key(42)
    perm = jax.random.permutation(rkey, n)
    mask_idx = perm[:num_mask]
    mvec = jnp.zeros((n,), jnp.float32).at[mask_idx].set(1.0)
    w = jax.random.normal(jax.random.fold_in(rkey, 1), (x.shape[1], d),
                          dtype=jnp.float32) * 0.01
    wt = w.T

    nchunk = -(-e // (_NW * _CH))
    nchunk = (nchunk + 1) // 2 * 2
    tot = _NW * nchunk * _CH
    junk = jnp.int32(n)
    row = edge_index[0].astype(jnp.int32)
    col = edge_index[1].astype(jnp.int32)
    pad = tot - e
    rc = jnp.stack([
        jnp.concatenate([row, jnp.full((pad,), junk)]),
        jnp.concatenate([col, jnp.full((pad,), junk)]),
    ])
    idx_p = rc.reshape(2, _NW, nchunk, _CH).transpose(1, 2, 0, 3)
    emb_pad = jnp.concatenate([embed, jnp.zeros((16, d), jnp.float32)], axis=0)

    mesh = plsc.VectorSubcoreMesh(core_axis_name="c", subcore_axis_name="s")
    sc_fn = pl.kernel(
        functools.partial(_sc_body, nchunk, d),
        out_type=[
            jax.ShapeDtypeStruct((_NC * _ACC, d), jnp.float32),
            jax.ShapeDtypeStruct((_NC, 2, _ACC), jnp.float32),
        ],
        mesh=mesh,
        scratch_types=[
            pltpu.VMEM((2, 2, _CH), jnp.int32),
            pltpu.VMEM((2, _CH, d), jnp.float32),
            pltpu.VMEM((16, d), jnp.float32),
            pltpu.VMEM((_CH,), jnp.float32),
            pltpu.VMEM((_RPT,), jnp.float32),
            pltpu.VMEM_SHARED((_ACC, d), jnp.float32),
            pltpu.VMEM_SHARED((_ACC,), jnp.float32),
            pltpu.VMEM_SHARED((_ACC,), jnp.float32),
            (pltpu.SemaphoreType.DMA,) * 2,
            (pltpu.SemaphoreType.DMA,) * 2,
        ],
    )
    acc_out, deg_out = sc_fn(idx_p, emb_pad)

    out = pl.pallas_call(
        functools.partial(_tc_body, num_mask, e),
        out_shape=jax.ShapeDtypeStruct((1, 1), jnp.float32),
    )(embed, acc_out.reshape(_NC, _ACC, d)[:, :n, :], deg_out[:, :, :n],
      mvec, wt)
    return out[0, 0]


# speedup vs baseline: 1.1300x; 1.1300x over previous
import functools

import jax
import jax.numpy as jnp
from jax import lax
from jax.experimental import pallas as pl
from jax.experimental.pallas import tpu as pltpu
from jax.experimental.pallas import tpu_sc as plsc

_MASK_RATIO = 0.5
_NEIGH_WEIGHT = 0.5

_NC = 2
_NS = 16
_NW = _NC * _NS
_CH = 128
_ACC = 10240
_RPT = _ACC // _NS


def _sc_body(nchunk, d, idx_ref, emb_ref, acc_out, deg_out,
             idx_g, buf, zrow, ones_l, zdeg, acc_s, deg_r_s, deg_c_s):
    cid = lax.axis_index("c")
    sid = lax.axis_index("s")
    tid = cid * _NS + sid

    z16 = jnp.zeros((16,), jnp.float32)
    o16 = jnp.full((16,), 1.0, jnp.float32)
    for i in range(16):
        for j in range(d // 16):
            zrow[i, pl.ds(j * 16, 16)] = z16
    for j in range(_CH // 16):
        ones_l[pl.ds(j * 16, 16)] = o16

    def _zero_zdeg(k, carry):
        zdeg[pl.ds(k * 16, 16)] = z16
        return carry

    lax.fori_loop(0, _RPT // 16, _zero_zdeg, 0)

    base = sid * _RPT
    pltpu.sync_copy(zdeg, deg_r_s.at[pl.ds(base, _RPT)])
    pltpu.sync_copy(zdeg, deg_c_s.at[pl.ds(base, _RPT)])
    for b in range(_RPT // 16):
        pltpu.sync_copy(zrow, acc_s.at[pl.ds(base + b * 16, 16)])
    plsc.subcore_barrier()

    def body(c, carry):
        pltpu.sync_copy(idx_ref.at[tid, c], idx_g)
        pltpu.sync_copy(emb_ref.at[idx_g.at[1]], buf)
        pltpu.sync_copy(ones_l, deg_r_s.at[idx_g.at[0]], add=True)
        pltpu.sync_copy(ones_l, deg_c_s.at[idx_g.at[1]], add=True)
        pltpu.sync_copy(buf, acc_s.at[idx_g.at[0]], add=True)
        return carry

    lax.fori_loop(0, nchunk, body, 0)
    plsc.subcore_barrier()

    pltpu.sync_copy(acc_s.at[pl.ds(base, _RPT)],
                    acc_out.at[pl.ds(cid * _ACC + base, _RPT)])
    pltpu.sync_copy(deg_r_s.at[pl.ds(base, _RPT)],
                    deg_out.at[cid, 0, pl.ds(base, _RPT)])
    pltpu.sync_copy(deg_c_s.at[pl.ds(base, _RPT)],
                    deg_out.at[cid, 1, pl.ds(base, _RPT)])


def _tc_body(num_mask, num_edges, emb_ref, acc_ref, deg_ref, mvec_ref, wt_ref,
             out_ref):
    emb = emb_ref[...]
    ns = acc_ref[0] + acc_ref[1]
    dr_raw = deg_ref[0, 0] + deg_ref[1, 0]
    dc = deg_ref[0, 1] + deg_ref[1, 1]
    mvec = mvec_ref[...]

    r = jnp.dot(emb, wt_ref[...], preferred_element_type=jnp.float32)
    nm = ns / jnp.maximum(dr_raw, 1.0)[:, None]
    dvec = r - nm
    recon_sum = jnp.sum(mvec * jnp.sum(dvec * dvec, axis=1))
    nrm = jnp.sum(emb * emb, axis=1)
    sq_sum = jnp.sum((dr_raw + dc) * nrm)
    dot_sum = jnp.sum(emb * ns)

    d = emb.shape[1]
    recon_loss = recon_sum / (num_mask * d)
    neigh_loss = (sq_sum - 2.0 * dot_sum) / num_edges
    total = recon_loss + _NEIGH_WEIGHT * neigh_loss
    out_ref[...] = total[None, None]


def kernel(x, edge_index, embed):
    n, d = embed.shape
    e = edge_index.shape[1]
    num_mask = max(1, int(_MASK_RATIO * n))

    rkey = jax.random.key(42)
    perm = jax.random.permutation(rkey, n)
    mask_idx = perm[:num_mask]
    mvec = jnp.zeros((n,), jnp.float32).at[mask_idx].set(1.0)
    w = jax.random.normal(jax.random.fold_in(rkey, 1), (x.shape[1], d),
                          dtype=jnp.float32) * 0.01
    wt = w.T

    nchunk = -(-e // (_NW * _CH))
    tot = _NW * nchunk * _CH
    junk = jnp.int32(n)
    row = edge_index[0].astype(jnp.int32)
    col = edge_index[1].astype(jnp.int32)
    pad = tot - e
    rc = jnp.stack([
        jnp.concatenate([row, jnp.full((pad,), junk)]),
        jnp.concatenate([col, jnp.full((pad,), junk)]),
    ])
    idx_p = rc.reshape(2, _NW, nchunk, _CH).transpose(1, 2, 0, 3)
    emb_pad = jnp.concatenate([embed, jnp.zeros((16, d), jnp.float32)], axis=0)

    mesh = plsc.VectorSubcoreMesh(core_axis_name="c", subcore_axis_name="s")
    sc_fn = pl.kernel(
        functools.partial(_sc_body, nchunk, d),
        out_type=[
            jax.ShapeDtypeStruct((_NC * _ACC, d), jnp.float32),
            jax.ShapeDtypeStruct((_NC, 2, _ACC), jnp.float32),
        ],
        mesh=mesh,
        scratch_types=[
            pltpu.VMEM((2, _CH), jnp.int32),
            pltpu.VMEM((_CH, d), jnp.float32),
            pltpu.VMEM((16, d), jnp.float32),
            pltpu.VMEM((_CH,), jnp.float32),
            pltpu.VMEM((_RPT,), jnp.float32),
            pltpu.VMEM_SHARED((_ACC, d), jnp.float32),
            pltpu.VMEM_SHARED((_ACC,), jnp.float32),
            pltpu.VMEM_SHARED((_ACC,), jnp.float32),
        ],
    )
    acc_out, deg_out = sc_fn(idx_p, emb_pad)

    out = pl.pallas_call(
        functools.partial(_tc_body, num_mask, e),
        out_shape=jax.ShapeDtypeStruct((1, 1), jnp.float32),
    )(embed, acc_out.reshape(_NC, _ACC, d)[:, :n, :], deg_out[:, :, :n],
      mvec, wt)
    return out[0, 0]
